# baseline (speedup 1.0000x reference)
import functools

import jax
import jax.numpy as jnp
from jax import lax
from jax.experimental import pallas as pl
from jax.experimental.pallas import tpu as pltpu
from jax.experimental.pallas import tpu_sc as plsc

N = 10000
E = 160000
F = 128
G = 64


CH = 128
NTILES = 16
CPT = 80
HALF = 40
NCHUNK_PAD = NTILES * CPT
E_PAD = NCHUNK_PAD * CH
ROWS_MAIN = 640
ROWS_LAST = N - 15 * ROWS_MAIN


def _tile_rowcopy(src, dst, sid):
    row0 = pl.multiple_of(sid * ROWS_MAIN, 8)

    @pl.when(sid < NTILES - 1)
    def _():
        pltpu.sync_copy(src.at[pl.ds(row0, ROWS_MAIN)],
                        dst.at[pl.ds(row0, ROWS_MAIN)])

    @pl.when(sid == NTILES - 1)
    def _():
        pltpu.sync_copy(src.at[pl.ds(15 * ROWS_MAIN, ROWS_LAST)],
                        dst.at[pl.ds(15 * ROWS_MAIN, ROWS_LAST)])

_sc_mesh = plsc.VectorSubcoreMesh(core_axis_name="c", subcore_axis_name="s")


@functools.partial(
    pl.kernel,
    out_type=(
        jax.ShapeDtypeStruct((N, F), jnp.float32),
        jax.ShapeDtypeStruct((N, F), jnp.float32),
    ),
    mesh=_sc_mesh,
    scratch_types=[
        pltpu.VMEM((HALF, CH), jnp.int32),
        pltpu.VMEM((HALF, CH), jnp.int32),
        pltpu.VMEM((HALF, CH), jnp.float32),
        pltpu.VMEM((2, CH, F), jnp.float32),
        pltpu.VMEM((2, CH), jnp.int32),
        pltpu.VMEM((2, CH), jnp.int32),
        pltpu.VMEM_SHARED((N, F), jnp.float32),
        pltpu.SemaphoreType.DMA,
        pltpu.SemaphoreType.DMA,
        pltpu.SemaphoreType.DMA,
        pltpu.SemaphoreType.DMA,
    ],
)
def _mp_kernel(ei0, ei1, ew, xl1, xl2, init1, init2, out1, out2,
               srcs_v, dsts_v, ews_v, rows_v, dst2_v, src2_v, acc,
               sg0, sg1, ss0, ss1):
    c = lax.axis_index("c")
    sid = lax.axis_index("s")

    @pl.when(c == 0)
    def _():
        _tile_rowcopy(init1, acc, sid)

    @pl.when(c == 1)
    def _():
        _tile_rowcopy(init2, acc, sid)

    plsc.subcore_barrier()

    sg = (sg0, sg1)
    ssm = (ss0, ss1)

    def run(src_hbm, dst_hbm, x_hbm):
        def stage_gather(t, b):
            for fb in range(CH // 16):
                sl = pl.ds(fb * 16, 16)
                src2_v[b, sl] = srcs_v[t, sl]
            pltpu.async_copy(x_hbm.at[src2_v.at[b]], rows_v.at[b], sg[b])

        def gwait(b):
            pltpu.make_async_copy(x_hbm.at[src2_v.at[b]], rows_v.at[b],
                                  sg[b]).wait()

        def pair_body(u, carry):
            for b in range(2):
                t = 2 * u + b
                gwait(b)

                @pl.when(t + 1 < HALF)
                def _(b=b, t=t):
                    stage_gather(t + 1, 1 - b)

                for fb in range(CH // 16):
                    sl = pl.ds(fb * 16, 16)
                    dst2_v[b, sl] = dsts_v[t, sl]

                def grp_body(g, rc, _t=t, _b=b):
                    g16 = pl.multiple_of(g * 16, 16)
                    wv = ews_v[_t, pl.ds(g16, 16)]
                    for j in range(16):
                        w = wv[j]
                        for fb in range(F // 16):
                            sl = pl.ds(fb * 16, 16)
                            rows_v[_b, g16 + j, sl] = (
                                rows_v[_b, g16 + j, sl] * w)
                    return rc

                lax.fori_loop(0, CH // 16, grp_body, 0)
                pltpu.sync_copy(rows_v.at[b], acc.at[dst2_v.at[b]],
                                add=True)
            return carry

        for h in range(CPT // HALF):
            base = pl.multiple_of(sid * CPT + h * HALF, 8)
            pltpu.sync_copy(src_hbm.at[pl.ds(base, HALF)], srcs_v)
            pltpu.sync_copy(dst_hbm.at[pl.ds(base, HALF)], dsts_v)
            pltpu.sync_copy(ew.at[pl.ds(base, HALF)], ews_v)
            stage_gather(0, 0)
            lax.fori_loop(0, HALF // 2, pair_body, 0)

    @pl.when(c == 0)
    def _():
        run(ei0, ei1, xl1)

    @pl.when(c == 1)
    def _():
        run(ei1, ei0, xl2)

    plsc.subcore_barrier()

    @pl.when(c == 0)
    def _():
        _tile_rowcopy(acc, out1, sid)

    @pl.when(c == 1)
    def _():
        _tile_rowcopy(acc, out2, sid)


BLK = 1000
NBLK = N // BLK


def _leaky(v):
    return jnp.where(v >= 0, v, 0.2 * v)


def _mm(a, w):
    return lax.dot_general(a, w, (((1,), (1,)), ((), ())),
                           preferred_element_type=jnp.float32)


def _pre_body(x_ref, c00_ref, c10_ref, o1_ref, o2_ref):
    xs = x_ref[...]
    o1_ref[...] = _mm(xs[:, :F], c00_ref[...])
    o2_ref[...] = _mm(xs[:, F:], c10_ref[...])


def _pre_call(x, c00, c10):
    return pl.pallas_call(
        _pre_body,
        grid=(NBLK,),
        in_specs=[
            pl.BlockSpec((BLK, 2 * F), lambda i: (i, 0)),
            pl.BlockSpec((F, F), lambda i: (0, 0)),
            pl.BlockSpec((F, F), lambda i: (0, 0)),
        ],
        out_specs=[
            pl.BlockSpec((BLK, F), lambda i: (i, 0)),
            pl.BlockSpec((BLK, F), lambda i: (i, 0)),
        ],
        out_shape=[
            jax.ShapeDtypeStruct((N, F), jnp.float32),
            jax.ShapeDtypeStruct((N, F), jnp.float32),
        ],
    )(x, c00, c10)


def _mid_body(a1_ref, a2_ref, s_ref, aw_ref, w11_ref, b1_ref, w13_ref,
              b3_ref, c0n_ref, c1n_ref, o1_ref, o2_ref):
    s = s_ref[...]
    x1s = _leaky(a1_ref[...] + s * aw_ref[0, 0])
    x1 = _leaky(_mm(x1s, w11_ref[...]) + b1_ref[...])
    o1_ref[...] = _mm(x1, c0n_ref[...])
    x2s = _leaky(a2_ref[...] + s * aw_ref[1, 0])
    x2 = _leaky(_mm(x2s, w13_ref[...]) + b3_ref[...])
    o2_ref[...] = _mm(x2, c1n_ref[...])


def _mid_call(a1, a2, s2, aw, w11, b1, w13, b3, c0n, c1n):
    wspec = pl.BlockSpec((F, F), lambda i: (0, 0))
    bspec = pl.BlockSpec((1, F), lambda i: (0, 0))
    nspec = pl.BlockSpec((BLK, F), lambda i: (i, 0))
    return pl.pallas_call(
        _mid_body,
        grid=(NBLK,),
        in_specs=[
            nspec, nspec,
            pl.BlockSpec((BLK, 1), lambda i: (i, 0)),
            pl.BlockSpec(memory_space=pltpu.SMEM),
            wspec, bspec, wspec, bspec, wspec, wspec,
        ],
        out_specs=[nspec, nspec],
        out_shape=[
            jax.ShapeDtypeStruct((N, F), jnp.float32),
            jax.ShapeDtypeStruct((N, F), jnp.float32),
        ],
    )(a1, a2, s2, aw, w11, b1, w13, b3, c0n, c1n)


def _post_body(a1_ref, a2_ref, s_ref, batch_ref, aw_ref, w11_ref, b1_ref,
               w13_ref, b3_ref, t0_ref, t1_ref, t2_ref, t2b_ref,
               xc_ref, gsum_ref):
    i = pl.program_id(0)
    s = s_ref[...]
    x1s = _leaky(a1_ref[...] + s * aw_ref[0, 0])
    x1 = _leaky(_mm(x1s, w11_ref[...]) + b1_ref[...])
    x2s = _leaky(a2_ref[...] + s * aw_ref[1, 0])
    x2 = _leaky(_mm(x2s, w13_ref[...]) + b3_ref[...])
    h1 = _mm(x1, t0_ref[...])
    h2 = _mm(x2, t1_ref[...])
    t2 = t2_ref[...]
    xc = _leaky(_mm(h1, t2[:, :F]) + _mm(h2, t2[:, F:]) + t2b_ref[...])
    xc_ref[...] = xc
    sel = (s == 1.0).astype(jnp.float32)
    y = xc * sel
    oh = (batch_ref[...] ==
          lax.broadcasted_iota(jnp.int32, (BLK, G), 1)).astype(jnp.float32)
    part = lax.dot_general(oh, y, (((0,), (0,)), ((), ())),
                           preferred_element_type=jnp.float32)

    @pl.when(i == 0)
    def _():
        gsum_ref[...] = jnp.zeros_like(gsum_ref)

    gsum_ref[...] += part


def _post_call(a1, a2, s2, batch2, aw, w11, b1, w13, b3, t0, t1, t2, t2b):
    wspec = pl.BlockSpec((F, F), lambda i: (0, 0))
    bspec = pl.BlockSpec((1, F), lambda i: (0, 0))
    nspec = pl.BlockSpec((BLK, F), lambda i: (i, 0))
    return pl.pallas_call(
        _post_body,
        grid=(NBLK,),
        in_specs=[
            nspec, nspec,
            pl.BlockSpec((BLK, 1), lambda i: (i, 0)),
            pl.BlockSpec((BLK, 1), lambda i: (i, 0)),
            pl.BlockSpec(memory_space=pltpu.SMEM),
            wspec, bspec, wspec, bspec,
            wspec, wspec,
            pl.BlockSpec((F, 2 * F), lambda i: (0, 0)),
            bspec,
        ],
        out_specs=[
            nspec,
            pl.BlockSpec((G, F), lambda i: (0, 0)),
        ],
        out_shape=[
            jax.ShapeDtypeStruct((N, F), jnp.float32),
            jax.ShapeDtypeStruct((G, F), jnp.float32),
        ],
    )(a1, a2, s2, batch2, aw, w11, b1, w13, b3, t0, t1, t2, t2b)


def _final_body(xc_ref, gsum_ref, batch_ref, d0_ref, d1_ref, d2_ref,
                d3_ref, d3b_ref, q_ref):
    oh = (batch_ref[...] ==
          lax.broadcasted_iota(jnp.int32, (BLK, G), 1)).astype(jnp.float32)
    x_s = lax.dot_general(oh, gsum_ref[...], (((1,), (0,)), ((), ())),
                          preferred_element_type=jnp.float32)
    p1 = _leaky(_mm(xc_ref[...], d0_ref[...]))
    p2 = _leaky(_mm(x_s, d1_ref[...]))
    d2 = d2_ref[...]
    x4 = _leaky(_mm(p1, d2[:, :F]) + _mm(p2, d2[:, F:]))
    q_ref[...] = (jnp.sum(x4 * d3_ref[...], axis=1, keepdims=True)
                  + d3b_ref[0, 0])


def _final_call(xc, gsum, batch2, d0, d1, d2, d3, d3b):
    wspec = pl.BlockSpec((F, F), lambda i: (0, 0))
    return pl.pallas_call(
        _final_body,
        grid=(NBLK,),
        in_specs=[
            pl.BlockSpec((BLK, F), lambda i: (i, 0)),
            pl.BlockSpec((G, F), lambda i: (0, 0)),
            pl.BlockSpec((BLK, 1), lambda i: (i, 0)),
            wspec, wspec,
            pl.BlockSpec((F, 2 * F), lambda i: (0, 0)),
            pl.BlockSpec((1, F), lambda i: (0, 0)),
            pl.BlockSpec(memory_space=pltpu.SMEM),
        ],
        out_specs=pl.BlockSpec((BLK, 1), lambda i: (i, 0)),
        out_shape=jax.ShapeDtypeStruct((N, 1), jnp.float32),
    )(xc, gsum, batch2, d0, d1, d2, d3, d3b)


@jax.jit
def _run(x, edge_index, edge_weights, batch, states, conv0_W, alpha0_w,
         alpha1_W, alpha1_b, conv1_W, alpha2_w, alpha3_W, alpha3_b,
         theta0_W, theta1_W, theta2_W, theta2_b, delta0_W, delta1_W,
         delta2_W, delta3_W, delta3_b):
    s = states.reshape(-1).astype(jnp.float32)
    s2 = s[:, None]
    batch2 = batch[:, None]
    padlen = E_PAD - E
    ipad = jnp.zeros((padlen,), jnp.int32)
    ei0 = jnp.concatenate([edge_index[0], ipad]).reshape(NCHUNK_PAD, CH)
    ei1 = jnp.concatenate([edge_index[1], ipad]).reshape(NCHUNK_PAD, CH)
    ewp = jnp.concatenate(
        [edge_weights, jnp.zeros((padlen,), jnp.float32)]).reshape(
            NCHUNK_PAD, CH)
    zeros = jnp.zeros((N, F), jnp.float32)

    xl1, xl2 = _pre_call(x, conv0_W[0], conv1_W[0])
    a1 = a2 = None
    for i in range(conv0_W.shape[0]):
        init1 = zeros if i == 0 else xl1
        init2 = zeros if i == 0 else xl2
        a1, a2 = _mp_kernel(ei0, ei1, ewp, xl1, xl2, init1, init2)
        aw = jnp.concatenate([alpha0_w[i, 0], alpha2_w[i, 0]])[:, None]
        if i < conv0_W.shape[0] - 1:
            xl1, xl2 = _mid_call(a1, a2, s2, aw, alpha1_W[i],
                                 alpha1_b[i][None], alpha3_W[i],
                                 alpha3_b[i][None], conv0_W[i + 1],
                                 conv1_W[i + 1])
        else:
            xc, gsum = _post_call(a1, a2, s2, batch2, aw, alpha1_W[i],
                                  alpha1_b[i][None], alpha3_W[i],
                                  alpha3_b[i][None], theta0_W, theta1_W,
                                  theta2_W, theta2_b[None])
    q = _final_call(xc, gsum, batch2, delta0_W, delta1_W, delta2_W,
                    delta3_W, delta3_b.reshape(1, 1))
    return q.reshape(-1)


def kernel(x, edge_index, edge_weights, batch, states, conv0_W, alpha0_w,
           alpha1_W, alpha1_b, conv1_W, alpha2_w, alpha3_W, alpha3_b,
           theta0_W, theta1_W, theta2_W, theta2_b, delta0_W, delta1_W,
           delta2_W, delta3_W, delta3_b):
    return _run(x, edge_index, edge_weights, batch, states, conv0_W,
                alpha0_w, alpha1_W, alpha1_b, conv1_W, alpha2_w, alpha3_W,
                alpha3_b, theta0_W, theta1_W, theta2_W, theta2_b, delta0_W,
                delta1_W, delta2_W, delta3_W, delta3_b)


# speedup vs baseline: 1.5547x; 1.5547x over previous
import functools

import jax
import jax.numpy as jnp
from jax import lax
from jax.experimental import pallas as pl
from jax.experimental.pallas import tpu as pltpu
from jax.experimental.pallas import tpu_sc as plsc

N = 10000
E = 160000
F = 128
G = 64


CH = 128
SUP = 8
EP = 157
NCHUNK_PAD = EP * SUP
E_PAD = NCHUNK_PAD * CH
NTILES = 16
KMAX = (EP + NTILES - 1) // NTILES
ROWS_MAIN = 640
ROWS_LAST = N - 15 * ROWS_MAIN


def _tile_rowcopy(src, dst, sid):
    row0 = pl.multiple_of(sid * ROWS_MAIN, 8)

    @pl.when(sid < NTILES - 1)
    def _():
        pltpu.sync_copy(src.at[pl.ds(row0, ROWS_MAIN)],
                        dst.at[pl.ds(row0, ROWS_MAIN)])

    @pl.when(sid == NTILES - 1)
    def _():
        pltpu.sync_copy(src.at[pl.ds(15 * ROWS_MAIN, ROWS_LAST)],
                        dst.at[pl.ds(15 * ROWS_MAIN, ROWS_LAST)])

_sc_mesh = plsc.VectorSubcoreMesh(core_axis_name="c", subcore_axis_name="s")


@functools.partial(
    pl.kernel,
    out_type=(
        jax.ShapeDtypeStruct((N, F), jnp.float32),
        jax.ShapeDtypeStruct((N, F), jnp.float32),
    ),
    mesh=_sc_mesh,
    scratch_types=[
        pltpu.VMEM((SUP, CH), jnp.int32),
        pltpu.VMEM((SUP, CH), jnp.int32),
        pltpu.VMEM((SUP, CH), jnp.float32),
        pltpu.VMEM((2, CH, F), jnp.float32),
        pltpu.VMEM_SHARED((N, F), jnp.float32),
        pltpu.SemaphoreType.DMA,
        pltpu.SemaphoreType.DMA,
        pltpu.SemaphoreType.DMA,
        pltpu.SemaphoreType.DMA,
    ],
)
def _mp_kernel(ei0, ei1, ew, xl1, xl2, init1, init2, out1, out2,
               srcs_v, dsts_v, ews_v, rows_v, acc,
               sg0, sg1, ss0, ss1):
    c = lax.axis_index("c")
    sid = lax.axis_index("s")

    @pl.when(c == 0)
    def _():
        _tile_rowcopy(init1, acc, sid)

    @pl.when(c == 1)
    def _():
        _tile_rowcopy(init2, acc, sid)

    plsc.subcore_barrier()

    sg = (sg0, sg1)
    ssm = (ss0, ss1)

    def run(src_hbm, dst_hbm, x_hbm):
        def super_body(k, carry):
            ss = k * NTILES + sid

            @pl.when(ss < EP)
            def _():
                row8 = pl.multiple_of(ss * SUP, SUP)
                pltpu.sync_copy(src_hbm.at[pl.ds(row8, SUP)], srcs_v)
                pltpu.sync_copy(dst_hbm.at[pl.ds(row8, SUP)], dsts_v)
                pltpu.sync_copy(ew.at[pl.ds(row8, SUP)], ews_v)

                gdesc = {}
                sdesc = {}
                gdesc[0] = pltpu.async_copy(
                    x_hbm.at[srcs_v.at[0]], rows_v.at[0], sg[0])
                for t in range(SUP):
                    b = t % 2
                    gdesc[t].wait()
                    if t >= 1:
                        sdesc[t - 1].wait()
                    if t + 1 < SUP:
                        gdesc[t + 1] = pltpu.async_copy(
                            x_hbm.at[srcs_v.at[t + 1]], rows_v.at[1 - b],
                            sg[1 - b])

                    def grp_body(g, rc, _t=t, _b=b):
                        g16 = pl.multiple_of(g * 16, 16)
                        wv = ews_v[_t, pl.ds(g16, 16)]
                        for j in range(16):
                            w = wv[j]
                            for fb in range(F // 16):
                                sl = pl.ds(fb * 16, 16)
                                rows_v[_b, g16 + j, sl] = (
                                    rows_v[_b, g16 + j, sl] * w)
                        return rc

                    lax.fori_loop(0, CH // 16, grp_body, 0)
                    sdesc[t] = pltpu.async_copy(
                        rows_v.at[b], acc.at[dsts_v.at[t]], ssm[b],
                        add=True)
                sdesc[SUP - 1].wait()

            return carry

        lax.fori_loop(0, KMAX, super_body, 0)

    @pl.when(c == 0)
    def _():
        run(ei0, ei1, xl1)

    @pl.when(c == 1)
    def _():
        run(ei1, ei0, xl2)

    plsc.subcore_barrier()

    @pl.when(c == 0)
    def _():
        _tile_rowcopy(acc, out1, sid)

    @pl.when(c == 1)
    def _():
        _tile_rowcopy(acc, out2, sid)


BLK = 1000
NBLK = N // BLK


def _leaky(v):
    return jnp.where(v >= 0, v, 0.2 * v)


def _mm(a, w):
    return lax.dot_general(a, w, (((1,), (1,)), ((), ())),
                           preferred_element_type=jnp.float32)


def _pre_body(x_ref, c00_ref, c10_ref, o1_ref, o2_ref):
    xs = x_ref[...]
    o1_ref[...] = _mm(xs[:, :F], c00_ref[...])
    o2_ref[...] = _mm(xs[:, F:], c10_ref[...])


def _pre_call(x, c00, c10):
    return pl.pallas_call(
        _pre_body,
        grid=(NBLK,),
        in_specs=[
            pl.BlockSpec((BLK, 2 * F), lambda i: (i, 0)),
            pl.BlockSpec((F, F), lambda i: (0, 0)),
            pl.BlockSpec((F, F), lambda i: (0, 0)),
        ],
        out_specs=[
            pl.BlockSpec((BLK, F), lambda i: (i, 0)),
            pl.BlockSpec((BLK, F), lambda i: (i, 0)),
        ],
        out_shape=[
            jax.ShapeDtypeStruct((N, F), jnp.float32),
            jax.ShapeDtypeStruct((N, F), jnp.float32),
        ],
    )(x, c00, c10)


def _mid_body(a1_ref, a2_ref, s_ref, aw_ref, w11_ref, b1_ref, w13_ref,
              b3_ref, c0n_ref, c1n_ref, o1_ref, o2_ref):
    s = s_ref[...]
    x1s = _leaky(a1_ref[...] + s * aw_ref[0, 0])
    x1 = _leaky(_mm(x1s, w11_ref[...]) + b1_ref[...])
    o1_ref[...] = _mm(x1, c0n_ref[...])
    x2s = _leaky(a2_ref[...] + s * aw_ref[1, 0])
    x2 = _leaky(_mm(x2s, w13_ref[...]) + b3_ref[...])
    o2_ref[...] = _mm(x2, c1n_ref[...])


def _mid_call(a1, a2, s2, aw, w11, b1, w13, b3, c0n, c1n):
    wspec = pl.BlockSpec((F, F), lambda i: (0, 0))
    bspec = pl.BlockSpec((1, F), lambda i: (0, 0))
    nspec = pl.BlockSpec((BLK, F), lambda i: (i, 0))
    return pl.pallas_call(
        _mid_body,
        grid=(NBLK,),
        in_specs=[
            nspec, nspec,
            pl.BlockSpec((BLK, 1), lambda i: (i, 0)),
            pl.BlockSpec(memory_space=pltpu.SMEM),
            wspec, bspec, wspec, bspec, wspec, wspec,
        ],
        out_specs=[nspec, nspec],
        out_shape=[
            jax.ShapeDtypeStruct((N, F), jnp.float32),
            jax.ShapeDtypeStruct((N, F), jnp.float32),
        ],
    )(a1, a2, s2, aw, w11, b1, w13, b3, c0n, c1n)


def _post_body(a1_ref, a2_ref, s_ref, batch_ref, aw_ref, w11_ref, b1_ref,
               w13_ref, b3_ref, t0_ref, t1_ref, t2_ref, t2b_ref,
               xc_ref, gsum_ref):
    i = pl.program_id(0)
    s = s_ref[...]
    x1s = _leaky(a1_ref[...] + s * aw_ref[0, 0])
    x1 = _leaky(_mm(x1s, w11_ref[...]) + b1_ref[...])
    x2s = _leaky(a2_ref[...] + s * aw_ref[1, 0])
    x2 = _leaky(_mm(x2s, w13_ref[...]) + b3_ref[...])
    h1 = _mm(x1, t0_ref[...])
    h2 = _mm(x2, t1_ref[...])
    t2 = t2_ref[...]
    xc = _leaky(_mm(h1, t2[:, :F]) + _mm(h2, t2[:, F:]) + t2b_ref[...])
    xc_ref[...] = xc
    sel = (s == 1.0).astype(jnp.float32)
    y = xc * sel
    oh = (batch_ref[...] ==
          lax.broadcasted_iota(jnp.int32, (BLK, G), 1)).astype(jnp.float32)
    part = lax.dot_general(oh, y, (((0,), (0,)), ((), ())),
                           preferred_element_type=jnp.float32)

    @pl.when(i == 0)
    def _():
        gsum_ref[...] = jnp.zeros_like(gsum_ref)

    gsum_ref[...] += part


def _post_call(a1, a2, s2, batch2, aw, w11, b1, w13, b3, t0, t1, t2, t2b):
    wspec = pl.BlockSpec((F, F), lambda i: (0, 0))
    bspec = pl.BlockSpec((1, F), lambda i: (0, 0))
    nspec = pl.BlockSpec((BLK, F), lambda i: (i, 0))
    return pl.pallas_call(
        _post_body,
        grid=(NBLK,),
        in_specs=[
            nspec, nspec,
            pl.BlockSpec((BLK, 1), lambda i: (i, 0)),
            pl.BlockSpec((BLK, 1), lambda i: (i, 0)),
            pl.BlockSpec(memory_space=pltpu.SMEM),
            wspec, bspec, wspec, bspec,
            wspec, wspec,
            pl.BlockSpec((F, 2 * F), lambda i: (0, 0)),
            bspec,
        ],
        out_specs=[
            nspec,
            pl.BlockSpec((G, F), lambda i: (0, 0)),
        ],
        out_shape=[
            jax.ShapeDtypeStruct((N, F), jnp.float32),
            jax.ShapeDtypeStruct((G, F), jnp.float32),
        ],
    )(a1, a2, s2, batch2, aw, w11, b1, w13, b3, t0, t1, t2, t2b)


def _final_body(xc_ref, gsum_ref, batch_ref, d0_ref, d1_ref, d2_ref,
                d3_ref, d3b_ref, q_ref):
    oh = (batch_ref[...] ==
          lax.broadcasted_iota(jnp.int32, (BLK, G), 1)).astype(jnp.float32)
    x_s = lax.dot_general(oh, gsum_ref[...], (((1,), (0,)), ((), ())),
                          preferred_element_type=jnp.float32)
    p1 = _leaky(_mm(xc_ref[...], d0_ref[...]))
    p2 = _leaky(_mm(x_s, d1_ref[...]))
    d2 = d2_ref[...]
    x4 = _leaky(_mm(p1, d2[:, :F]) + _mm(p2, d2[:, F:]))
    q_ref[...] = (jnp.sum(x4 * d3_ref[...], axis=1, keepdims=True)
                  + d3b_ref[0, 0])


def _final_call(xc, gsum, batch2, d0, d1, d2, d3, d3b):
    wspec = pl.BlockSpec((F, F), lambda i: (0, 0))
    return pl.pallas_call(
        _final_body,
        grid=(NBLK,),
        in_specs=[
            pl.BlockSpec((BLK, F), lambda i: (i, 0)),
            pl.BlockSpec((G, F), lambda i: (0, 0)),
            pl.BlockSpec((BLK, 1), lambda i: (i, 0)),
            wspec, wspec,
            pl.BlockSpec((F, 2 * F), lambda i: (0, 0)),
            pl.BlockSpec((1, F), lambda i: (0, 0)),
            pl.BlockSpec(memory_space=pltpu.SMEM),
        ],
        out_specs=pl.BlockSpec((BLK, 1), lambda i: (i, 0)),
        out_shape=jax.ShapeDtypeStruct((N, 1), jnp.float32),
    )(xc, gsum, batch2, d0, d1, d2, d3, d3b)


@jax.jit
def _run(x, edge_index, edge_weights, batch, states, conv0_W, alpha0_w,
         alpha1_W, alpha1_b, conv1_W, alpha2_w, alpha3_W, alpha3_b,
         theta0_W, theta1_W, theta2_W, theta2_b, delta0_W, delta1_W,
         delta2_W, delta3_W, delta3_b):
    s = states.reshape(-1).astype(jnp.float32)
    s2 = s[:, None]
    batch2 = batch[:, None]
    padlen = E_PAD - E
    ipad = jnp.zeros((padlen,), jnp.int32)
    ei0 = jnp.concatenate([edge_index[0], ipad]).reshape(NCHUNK_PAD, CH)
    ei1 = jnp.concatenate([edge_index[1], ipad]).reshape(NCHUNK_PAD, CH)
    ewp = jnp.concatenate(
        [edge_weights, jnp.zeros((padlen,), jnp.float32)]).reshape(
            NCHUNK_PAD, CH)
    zeros = jnp.zeros((N, F), jnp.float32)

    xl1, xl2 = _pre_call(x, conv0_W[0], conv1_W[0])
    a1 = a2 = None
    for i in range(conv0_W.shape[0]):
        init1 = zeros if i == 0 else xl1
        init2 = zeros if i == 0 else xl2
        a1, a2 = _mp_kernel(ei0, ei1, ewp, xl1, xl2, init1, init2)
        aw = jnp.concatenate([alpha0_w[i, 0], alpha2_w[i, 0]])[:, None]
        if i < conv0_W.shape[0] - 1:
            xl1, xl2 = _mid_call(a1, a2, s2, aw, alpha1_W[i],
                                 alpha1_b[i][None], alpha3_W[i],
                                 alpha3_b[i][None], conv0_W[i + 1],
                                 conv1_W[i + 1])
        else:
            xc, gsum = _post_call(a1, a2, s2, batch2, aw, alpha1_W[i],
                                  alpha1_b[i][None], alpha3_W[i],
                                  alpha3_b[i][None], theta0_W, theta1_W,
                                  theta2_W, theta2_b[None])
    q = _final_call(xc, gsum, batch2, delta0_W, delta1_W, delta2_W,
                    delta3_W, delta3_b.reshape(1, 1))
    return q.reshape(-1)


def kernel(x, edge_index, edge_weights, batch, states, conv0_W, alpha0_w,
           alpha1_W, alpha1_b, conv1_W, alpha2_w, alpha3_W, alpha3_b,
           theta0_W, theta1_W, theta2_W, theta2_b, delta0_W, delta1_W,
           delta2_W, delta3_W, delta3_b):
    return _run(x, edge_index, edge_weights, batch, states, conv0_W,
                alpha0_w, alpha1_W, alpha1_b, conv1_W, alpha2_w, alpha3_W,
                alpha3_b, theta0_W, theta1_W, theta2_W, theta2_b, delta0_W,
                delta1_W, delta2_W, delta3_W, delta3_b)
